# baseline (device time: 31489 ns/iter reference)
import jax
import jax.numpy as jnp
from jax import lax
from jax.experimental import pallas as pl
from jax.experimental.pallas import tpu as pltpu

N_DEV = 4


def kernel(x, Win0, Wout0, Win1, Wout1, Win2, Wout2):
    B, D = x.shape
    H = Win0.shape[1]
    HB = B // 2
    rows = B // N_DEV

    def body(x_ref, win0, wout0, win1, wout1, win2, wout2, out_ref,
             winbuf, woutbuf, win_bf, ar_buf, rs_stage, rs_buf,
             win_sem, wout_sem,
             ar_send_sems, ar_recv_sems, rs_send_sems, rs_recv_sems):
        my = lax.axis_index("i")
        wins = [win0, win1, win2]
        wouts = [wout0, wout1, wout2]
        win_dma = [pltpu.make_async_copy(wins[r], winbuf, win_sem)
                   for r in range(3)]
        wout_dma = [pltpu.make_async_copy(wouts[r], woutbuf, wout_sem)
                    for r in range(3)]

        win_dma[0].start()
        wout_dma[0].start()

        barrier_sem = pltpu.get_barrier_semaphore()
        for k in range(1, N_DEV):
            pl.semaphore_signal(
                barrier_sem, inc=1,
                device_id=((my + k) % N_DEV,),
                device_id_type=pl.DeviceIdType.MESH,
            )
        pl.semaphore_wait(barrier_sem, N_DEV - 1)

        def stage_weights(r):
            win_dma[r].wait()
            win_bf[...] = winbuf[...].astype(jnp.bfloat16)
            if r + 1 < 3:
                win_dma[r + 1].start()
            wout_dma[r].wait()

        started_wout = [False, False, False]

        def mlp_half(xh, r):
            h = jnp.dot(xh, win_bf[...], preferred_element_type=jnp.float32)
            h = jnp.maximum(h, 0.0).astype(jnp.bfloat16)
            p = jnp.dot(h, woutbuf[...].astype(jnp.bfloat16),
                        preferred_element_type=jnp.float32)
            if r + 1 < 3 and not started_wout[r]:
                wout_dma[r + 1].start()
                started_wout[r] = True
            return p

        def ar_start(r, half, p):
            ar_buf[r, half, 0] = p.astype(jnp.bfloat16)
            rdmas = []
            for k in range(1, N_DEV):
                rdma = pltpu.make_async_remote_copy(
                    src_ref=ar_buf.at[r, half, 0],
                    dst_ref=ar_buf.at[r, half, k],
                    send_sem=ar_send_sems.at[r, half, k],
                    recv_sem=ar_recv_sems.at[r, half, k],
                    device_id=((my + k) % N_DEV,),
                    device_id_type=pl.DeviceIdType.MESH,
                )
                rdma.start()
                rdmas.append(rdma)
            return rdmas

        def ar_finish(r, half, p, rdmas):
            for rdma in rdmas:
                rdma.wait_recv()
            total = p
            for k in range(1, N_DEV):
                total = total + ar_buf[r, half, k].astype(jnp.float32)
            for rdma in rdmas:
                rdma.wait_send()
            return total.astype(jnp.bfloat16)

        stage_weights(0)
        xa = x_ref[pl.ds(0, HB)].astype(jnp.bfloat16)
        xb = x_ref[pl.ds(HB, HB)].astype(jnp.bfloat16)

        pa = mlp_half(xa, 0)
        ar_a = ar_start(0, 0, pa)
        pb = mlp_half(xb, 0)
        ar_b = ar_start(0, 1, pb)
        stage_weights(1)

        xa = ar_finish(0, 0, pa, ar_a)
        pa = mlp_half(xa, 1)
        ar_a = ar_start(1, 0, pa)
        xb = ar_finish(0, 1, pb, ar_b)
        pb = mlp_half(xb, 1)
        ar_b = ar_start(1, 1, pb)
        stage_weights(2)

        xa = ar_finish(1, 0, pa, ar_a)
        rs_stage[pl.ds(0, HB)] = mlp_half(xa, 2).astype(jnp.bfloat16)
        xb = ar_finish(1, 1, pb, ar_b)
        rs_stage[pl.ds(HB, HB)] = mlp_half(xb, 2).astype(jnp.bfloat16)

        rs_rdmas = []
        for k in range(1, N_DEV):
            dest = (my + k) % N_DEV
            rdma = pltpu.make_async_remote_copy(
                src_ref=rs_stage.at[pl.ds(dest * rows, rows)],
                dst_ref=rs_buf.at[k],
                send_sem=rs_send_sems.at[k],
                recv_sem=rs_recv_sems.at[k],
                device_id=(dest,),
                device_id_type=pl.DeviceIdType.MESH,
            )
            rdma.start()
            rs_rdmas.append(rdma)
        for rdma in rs_rdmas:
            rdma.wait_recv()
        total = rs_stage[pl.ds(my * rows, rows)].astype(jnp.float32)
        for k in range(1, N_DEV):
            total = total + rs_buf[k].astype(jnp.float32)
        for rdma in rs_rdmas:
            rdma.wait_send()
        out_ref[...] = total

    return pl.pallas_call(
        body,
        out_shape=jax.ShapeDtypeStruct((rows, D), jnp.float32),
        in_specs=[pl.BlockSpec(memory_space=pltpu.VMEM)]
        + [pl.BlockSpec(memory_space=pl.ANY)] * 6,
        out_specs=pl.BlockSpec(memory_space=pltpu.VMEM),
        scratch_shapes=[
            pltpu.VMEM((D, H), jnp.float32),
            pltpu.VMEM((H, D), jnp.float32),
            pltpu.VMEM((D, H), jnp.bfloat16),
            pltpu.VMEM((2, 2, N_DEV, HB, D), jnp.bfloat16),
            pltpu.VMEM((B, D), jnp.bfloat16),
            pltpu.VMEM((N_DEV, rows, D), jnp.bfloat16),
            pltpu.SemaphoreType.DMA,
            pltpu.SemaphoreType.DMA,
            pltpu.SemaphoreType.DMA((2, 2, N_DEV)),
            pltpu.SemaphoreType.DMA((2, 2, N_DEV)),
            pltpu.SemaphoreType.DMA((N_DEV,)),
            pltpu.SemaphoreType.DMA((N_DEV,)),
        ],
        compiler_params=pltpu.CompilerParams(collective_id=0),
    )(x, Win0, Wout0, Win1, Wout1, Win2, Wout2)
